# baseline (speedup 1.0000x reference)
import jax
import jax.numpy as jnp
from jax.experimental import pallas as pl


def _copy_block(w_ref, o_ref):
    o_ref[...] = w_ref[...]


def kernel(x, weight):
    seq_len = x.shape[1]
    _, max_len, d_model = weight.shape
    block = 512
    assert seq_len % block == 0
    grid = seq_len // block
    return pl.pallas_call(
        _copy_block,
        grid=(grid,),
        in_specs=[pl.BlockSpec((1, block, d_model), lambda i: (0, i, 0))],
        out_specs=pl.BlockSpec((1, block, d_model), lambda i: (0, i, 0)),
        out_shape=jax.ShapeDtypeStruct((1, seq_len, d_model), weight.dtype),
    )(weight)


# speedup vs baseline: 1.1059x; 1.1059x over previous
import jax
import jax.numpy as jnp
from jax.experimental import pallas as pl
from jax.experimental.pallas import tpu as pltpu

_BLOCK = 1024


def _copy_block(w_ref, o_ref):
    o_ref[...] = w_ref[...]


def kernel(x, weight):
    seq_len = x.shape[1]
    d_model = weight.shape[2]
    grid = seq_len // _BLOCK
    return pl.pallas_call(
        _copy_block,
        grid=(grid,),
        in_specs=[pl.BlockSpec((1, _BLOCK, d_model), lambda i: (0, i, 0))],
        out_specs=pl.BlockSpec((1, _BLOCK, d_model), lambda i: (0, i, 0)),
        out_shape=jax.ShapeDtypeStruct((1, seq_len, d_model), weight.dtype),
        compiler_params=pltpu.CompilerParams(
            dimension_semantics=("parallel",),
        ),
    )(weight)


# speedup vs baseline: 1.2270x; 1.1095x over previous
import jax
import jax.numpy as jnp
from jax.experimental import pallas as pl
from jax.experimental.pallas import tpu as pltpu

_BLOCK = 2048


def _copy_block(w_ref, o_ref):
    o_ref[...] = w_ref[...]


def kernel(x, weight):
    seq_len = x.shape[1]
    d_model = weight.shape[2]
    grid = seq_len // _BLOCK
    return pl.pallas_call(
        _copy_block,
        grid=(grid,),
        in_specs=[pl.BlockSpec((1, _BLOCK, d_model), lambda i: (0, i, 0))],
        out_specs=pl.BlockSpec((1, _BLOCK, d_model), lambda i: (0, i, 0)),
        out_shape=jax.ShapeDtypeStruct((1, seq_len, d_model), weight.dtype),
        compiler_params=pltpu.CompilerParams(
            dimension_semantics=("parallel",),
        ),
    )(weight)
